# baseline (speedup 1.0000x reference)
import functools

import jax
import jax.numpy as jnp
from jax.experimental import pallas as pl
from jax.experimental.pallas import tpu as pltpu


def _fused_body(h_ref, m_ref, w_ref, b_ref, g_ref, be_ref, o_ref, *, eps):
    h = h_ref[...].astype(jnp.float32)
    m = m_ref[...].astype(jnp.float32)
    sums = jnp.sum(h * m[:, :, None], axis=1)
    cnt = jnp.sum(m, axis=1, keepdims=True)
    pooled = sums / jnp.sqrt(cnt)
    y = jax.lax.dot_general(pooled, w_ref[...].astype(jnp.float32),
                            (((1,), (1,)), ((), ())),
                            preferred_element_type=jnp.float32)
    y = y + b_ref[...].astype(jnp.float32)[None, :]
    mean = jnp.mean(y, axis=-1, keepdims=True)
    var = jnp.mean(jnp.square(y - mean), axis=-1, keepdims=True)
    y = (y - mean) * jax.lax.rsqrt(var + eps)
    y = (y * g_ref[...].astype(jnp.float32)[None, :]
         + be_ref[...].astype(jnp.float32)[None, :])
    o_ref[...] = y[:, None, :].astype(o_ref.dtype)


def _largest_divisor_leq(n, cap):
    cap = max(1, min(n, int(cap)))
    for t in range(cap, 0, -1):
        if n % t == 0:
            return t
    return 1


def kernel(hidden_states, mask, weight, bias, gamma, beta):
    B, S, H = hidden_states.shape
    if mask is None:
        mask = jnp.ones((B, S), dtype=hidden_states.dtype)
    itemsize = jnp.dtype(hidden_states.dtype).itemsize

    tile_budget = 6 << 20
    TB = _largest_divisor_leq(B, max(1, tile_budget // (S * H * itemsize)))
    num_b = B // TB

    h_spec = pl.BlockSpec((TB, S, H), lambda i: (i, 0, 0))
    mask = mask.reshape(B, 1, S)
    m_spec = pl.BlockSpec((TB, None, S), lambda i: (i, 0, 0))
    o_spec = pl.BlockSpec((TB, 1, H), lambda i: (i, 0, 0))
    c_spec = pl.BlockSpec(memory_space=pltpu.MemorySpace.VMEM)

    return pl.pallas_call(
        functools.partial(_fused_body, eps=1e-12),
        out_shape=jax.ShapeDtypeStruct((B, 1, H), jnp.float32),
        grid=(num_b,),
        in_specs=[h_spec, m_spec, c_spec, c_spec, c_spec, c_spec],
        out_specs=o_spec,
        compiler_params=pltpu.CompilerParams(
            dimension_semantics=("parallel",),
            vmem_limit_bytes=56 << 20),
    )(hidden_states, mask, weight, bias, gamma, beta)


# speedup vs baseline: 1.1070x; 1.1070x over previous
import functools

import jax
import jax.numpy as jnp
from jax.experimental import pallas as pl
from jax.experimental.pallas import tpu as pltpu


def _fused_body(h_ref, m_ref, w_ref, b_ref, g_ref, be_ref, o_ref, *, eps):
    h = h_ref[...].astype(jnp.float32)
    m = m_ref[...].astype(jnp.float32)
    sums = jnp.sum(h * m[:, :, None], axis=1)
    cnt = jnp.sum(m, axis=1, keepdims=True)
    pooled = sums / jnp.sqrt(cnt)
    y = jax.lax.dot_general(pooled, w_ref[...].astype(jnp.float32),
                            (((1,), (1,)), ((), ())),
                            preferred_element_type=jnp.float32)
    y = y + b_ref[...].astype(jnp.float32)[None, :]
    mean = jnp.mean(y, axis=-1, keepdims=True)
    var = jnp.mean(jnp.square(y - mean), axis=-1, keepdims=True)
    y = (y - mean) * jax.lax.rsqrt(var + eps)
    y = (y * g_ref[...].astype(jnp.float32)[None, :]
         + be_ref[...].astype(jnp.float32)[None, :])
    o_ref[...] = y[:, None, :].astype(o_ref.dtype)


def _largest_divisor_leq(n, cap):
    cap = max(1, min(n, int(cap)))
    for t in range(cap, 0, -1):
        if n % t == 0:
            return t
    return 1


def kernel(hidden_states, mask, weight, bias, gamma, beta):
    B, S, H = hidden_states.shape
    if mask is None:
        mask = jnp.ones((B, S), dtype=hidden_states.dtype)
    itemsize = jnp.dtype(hidden_states.dtype).itemsize

    tile_budget = 12 << 20
    TB = _largest_divisor_leq(B, max(1, tile_budget // (S * H * itemsize)))
    num_b = B // TB

    h_spec = pl.BlockSpec((TB, S, H), lambda i: (i, 0, 0))
    m_spec = pl.BlockSpec((TB, S), lambda i: (i, 0))
    o_spec = pl.BlockSpec((TB, 1, H), lambda i: (i, 0, 0))
    c_spec = pl.BlockSpec(memory_space=pltpu.MemorySpace.VMEM)

    return pl.pallas_call(
        functools.partial(_fused_body, eps=1e-12),
        out_shape=jax.ShapeDtypeStruct((B, 1, H), jnp.float32),
        grid=(num_b,),
        in_specs=[h_spec, m_spec, c_spec, c_spec, c_spec, c_spec],
        out_specs=o_spec,
        compiler_params=pltpu.CompilerParams(
            dimension_semantics=("parallel",),
            vmem_limit_bytes=56 << 20),
    )(hidden_states, mask, weight, bias, gamma, beta)
